# baseline (speedup 1.0000x reference)
import functools

import jax
import jax.numpy as jnp
from jax import lax
from jax.experimental import pallas as pl
from jax.experimental.pallas import tpu as pltpu
from jax.experimental.pallas import tpu_sc as plsc

D = 64
NUM_CORES = 2
NUM_SUBCORES = 16
LANES = 16
CH = 128


def _gather_kernel(n_rows, b_rows):
    cols_sc = D // NUM_CORES
    i_per_t = b_rows // NUM_SUBCORES
    n_desc = i_per_t // CH
    s_len = (n_rows // 3) // 128 * 128
    lens = (s_len, s_len, n_rows - 2 * s_len)
    starts = (0, s_len, 2 * s_len)
    buf_len = max(lens)
    n_pairs = cols_sc // 2

    @functools.partial(
        pl.kernel,
        mesh=plsc.VectorSubcoreMesh(core_axis_name="c", subcore_axis_name="s"),
        out_type=jax.ShapeDtypeStruct((D * b_rows,), jnp.float32),
        scratch_types=[
            pltpu.VMEM((cols_sc, i_per_t), jnp.int32),
            pltpu.VMEM((cols_sc, i_per_t), jnp.float32),
            pltpu.VMEM((i_per_t,), jnp.int32),
            pltpu.VMEM((i_per_t,), jnp.int32),
            pltpu.VMEM((i_per_t,), jnp.int32),
            pltpu.VMEM((i_per_t,), jnp.float32),
            pltpu.VMEM((i_per_t,), jnp.float32),
            pltpu.VMEM((i_per_t,), jnp.float32),
            pltpu.VMEM_SHARED((buf_len,), jnp.float32),
            pltpu.VMEM_SHARED((buf_len,), jnp.float32),
            pltpu.SemaphoreType.DMA,
            pltpu.SemaphoreType.DMA,
        ],
    )
    def k(tbl_hbm, idx_hbm, out_hbm, idx_v, out_v, cl0, cl1, cl2,
          g0, g1, g2, buf0, buf1, sem_stage, sem_g):
        c = lax.axis_index("c")
        s = lax.axis_index("s")
        j0 = c * cols_sc
        t0 = s * i_per_t

        tbl_seg = [tbl_hbm.at[:, pl.ds(starts[i], lens[i])] for i in range(3)]
        bufs = (buf0, buf1)
        cls = (cl0, cl1, cl2)
        gs = (g0, g1, g2)

        icopies = []
        for jl in range(cols_sc):
            icopies.append(
                pltpu.async_copy(
                    idx_hbm.at[pl.ds((j0 + jl) * b_rows + t0, i_per_t)],
                    idx_v.at[jl],
                    sem_g,
                )
            )
        for cp in icopies:
            cp.wait()

        @pl.when(s == 0)
        def _stage_first():
            pltpu.async_copy(
                tbl_seg[0].at[j0], buf0.at[pl.ds(0, lens[0])], sem_stage
            ).wait()

        plsc.subcore_barrier()

        def fire_stage(row, seg, buf):
            pltpu.async_copy(
                tbl_seg[seg].at[row], buf.at[pl.ds(0, lens[seg])], sem_stage
            )

        def drain_stage(row, seg, buf):
            pltpu.make_async_copy(
                tbl_seg[seg].at[row], buf.at[pl.ds(0, lens[seg])], sem_stage
            ).wait()

        def gather_seg(jl, seg, buf):
            def fire(kd, carry):
                sl = pl.ds(kd * CH, CH)
                pltpu.async_copy(
                    buf.at[cls[seg].at[sl]], gs[seg].at[sl], sem_g
                )
                return carry

            lax.fori_loop(0, n_desc, fire, 0, unroll=False)
            pltpu.make_async_copy(
                out_hbm.at[pl.ds(0, i_per_t)], gs[seg], sem_g
            ).wait()

        def clamp_all(jl):
            def body(i, carry):
                sl = pl.ds(i * LANES, LANES)
                iv = idx_v[jl, sl]
                for seg in range(3):
                    cls[seg][sl] = jnp.minimum(
                        jnp.maximum(iv - starts[seg], 0), lens[seg] - 1
                    )
                return carry

            lax.fori_loop(0, i_per_t // LANES, body, 0, unroll=False)

        def select_out(jl):
            def body(i, carry):
                sl = pl.ds(i * LANES, LANES)
                iv = idx_v[jl, sl]
                out_v[jl, sl] = jnp.where(
                    iv < starts[1],
                    g0[sl],
                    jnp.where(iv < starts[2], g1[sl], g2[sl]),
                )
                return carry

            lax.fori_loop(0, i_per_t // LANES, body, 0, unroll=False)

        def do_column(jl, par, last):
            b = lambda seg: bufs[(par + seg) % 2]
            clamp_all(jl)

            @pl.when(s == 0)
            def _f1():
                fire_stage(j0 + jl, 1, b(1))

            gather_seg(jl, 0, b(0))

            @pl.when(s == 0)
            def _d1():
                drain_stage(j0 + jl, 1, b(1))

            plsc.subcore_barrier()

            @pl.when(s == 0)
            def _f2():
                fire_stage(j0 + jl, 2, b(2))

            gather_seg(jl, 1, b(1))

            @pl.when(s == 0)
            def _d2():
                drain_stage(j0 + jl, 2, b(2))

            plsc.subcore_barrier()

            @pl.when((s == 0) & jnp.logical_not(last))
            def _f0():
                fire_stage(j0 + jl + 1, 0, b(3))

            gather_seg(jl, 2, b(2))
            select_out(jl)

            @pl.when((s == 0) & jnp.logical_not(last))
            def _d0():
                drain_stage(j0 + jl + 1, 0, b(3))

            plsc.subcore_barrier()

        def per_pair(p, carry):
            do_column(2 * p, 0, jnp.bool_(False))
            do_column(2 * p + 1, 1, p + 1 >= n_pairs)
            return carry

        lax.fori_loop(0, n_pairs, per_pair, 0, unroll=False)

        wcopies = []
        for jl in range(cols_sc):
            wcopies.append(
                pltpu.async_copy(
                    out_v.at[jl],
                    out_hbm.at[pl.ds((j0 + jl) * b_rows + t0, i_per_t)],
                    sem_g,
                )
            )
        for cp in wcopies:
            cp.wait()

    return k


def kernel(self_tensor, indices):
    n, d = self_tensor.shape
    b, d2 = indices.shape
    assert d == D and d2 == D
    idx_cm = indices.T.reshape(d * b)
    out_cm = _gather_kernel(n, b)(self_tensor.T, idx_cm)
    return out_cm.reshape(d, b).T


# speedup vs baseline: 2.5594x; 2.5594x over previous
import functools

import jax
import jax.numpy as jnp
from jax import lax
from jax.experimental import pallas as pl
from jax.experimental.pallas import tpu as pltpu
from jax.experimental.pallas import tpu_sc as plsc

D = 64
NUM_CORES = 2
NUM_SUBCORES = 16
LANES = 16
CH = 128


def _gather_kernel(n_rows, b_rows):
    cols_sc = D // NUM_CORES
    i_per_t = b_rows // NUM_SUBCORES
    n_desc = i_per_t // CH

    @functools.partial(
        pl.kernel,
        mesh=plsc.VectorSubcoreMesh(core_axis_name="c", subcore_axis_name="s"),
        out_type=jax.ShapeDtypeStruct((D * b_rows,), jnp.float32),
        scratch_types=[
            pltpu.VMEM((cols_sc, i_per_t), jnp.int32),
            pltpu.VMEM((cols_sc, i_per_t), jnp.float32),
            pltpu.VMEM_SHARED((n_rows,), jnp.float32),
            pltpu.SemaphoreType.DMA,
            pltpu.SemaphoreType.DMA,
        ],
    )
    def k(tbl_hbm, idx_hbm, out_hbm, idx_v, out_v, col_sh, sem_stage, sem_g):
        c = lax.axis_index("c")
        s = lax.axis_index("s")
        j0 = c * cols_sc
        t0 = s * i_per_t

        icopies = []
        for jl in range(cols_sc):
            icopies.append(
                pltpu.async_copy(
                    idx_hbm.at[pl.ds((j0 + jl) * b_rows + t0, i_per_t)],
                    idx_v.at[jl],
                    sem_g,
                )
            )
        for cp in icopies:
            cp.wait()

        def per_column(jl, carry):
            @pl.when(s == 0)
            def _stage():
                pltpu.async_copy(
                    tbl_hbm.at[j0 + jl], col_sh, sem_stage
                ).wait()

            plsc.subcore_barrier()

            copies = []
            for kd in range(n_desc):
                copies.append(
                    pltpu.async_copy(
                        col_sh.at[idx_v.at[jl, pl.ds(kd * CH, CH)]],
                        out_v.at[jl, pl.ds(kd * CH, CH)],
                        sem_g,
                    )
                )
            for cp in copies:
                cp.wait()

            plsc.subcore_barrier()
            return carry

        lax.fori_loop(0, cols_sc, per_column, 0, unroll=False)

        wcopies = []
        for jl in range(cols_sc):
            wcopies.append(
                pltpu.async_copy(
                    out_v.at[jl],
                    out_hbm.at[pl.ds((j0 + jl) * b_rows + t0, i_per_t)],
                    sem_g,
                )
            )
        for cp in wcopies:
            cp.wait()

    return k


def kernel(self_tensor, indices):
    n, d = self_tensor.shape
    b, d2 = indices.shape
    assert d == D and d2 == D
    idx_cm = indices.T.reshape(d * b)
    out_cm = _gather_kernel(n, b)(self_tensor.T, idx_cm)
    return out_cm.reshape(d, b).T


# speedup vs baseline: 2.6815x; 1.0477x over previous
import functools

import jax
import jax.numpy as jnp
from jax import lax
from jax.experimental import pallas as pl
from jax.experimental.pallas import tpu as pltpu
from jax.experimental.pallas import tpu_sc as plsc

D = 64
NUM_CORES = 2
NUM_SUBCORES = 16
LANES = 16
CH = 128


def _gather_kernel(n_rows, b_rows):
    cols_sc = D // NUM_CORES
    i_per_t = b_rows // NUM_SUBCORES
    n_desc = i_per_t // CH

    @functools.partial(
        pl.kernel,
        mesh=plsc.VectorSubcoreMesh(core_axis_name="c", subcore_axis_name="s"),
        out_type=jax.ShapeDtypeStruct((D, b_rows), jnp.float32),
        scratch_types=[
            pltpu.VMEM((cols_sc, i_per_t), jnp.int32),
            pltpu.VMEM((cols_sc, i_per_t), jnp.float32),
            pltpu.VMEM_SHARED((n_rows,), jnp.float32),
            pltpu.SemaphoreType.DMA,
            pltpu.SemaphoreType.DMA,
        ],
    )
    def k(tbl_hbm, idx_hbm, out_hbm, idx_v, out_v, col_sh, sem_stage, sem_g):
        c = lax.axis_index("c")
        s = lax.axis_index("s")
        j0 = c * cols_sc
        t0 = s * i_per_t

        pltpu.sync_copy(
            idx_hbm.at[pl.ds(j0, cols_sc), pl.ds(t0, i_per_t)], idx_v
        )

        def per_column(jl, carry):
            @pl.when(s == 0)
            def _stage():
                pltpu.async_copy(
                    tbl_hbm.at[j0 + jl], col_sh, sem_stage
                ).wait()

            plsc.subcore_barrier()

            copies = []
            for kd in range(n_desc):
                copies.append(
                    pltpu.async_copy(
                        col_sh.at[idx_v.at[jl, pl.ds(kd * CH, CH)]],
                        out_v.at[jl, pl.ds(kd * CH, CH)],
                        sem_g,
                    )
                )
            for cp in copies:
                cp.wait()

            plsc.subcore_barrier()
            return carry

        lax.fori_loop(0, cols_sc, per_column, 0, unroll=False)

        pltpu.sync_copy(
            out_v, out_hbm.at[pl.ds(j0, cols_sc), pl.ds(t0, i_per_t)]
        )

    return k


def kernel(self_tensor, indices):
    n, d = self_tensor.shape
    b, d2 = indices.shape
    assert d == D and d2 == D
    out_t = _gather_kernel(n, b)(self_tensor.T, indices.T)
    return out_t.T
